# baseline (speedup 1.0000x reference)
import functools

import jax
import jax.numpy as jnp
from jax import lax
from jax.experimental import pallas as pl
from jax.experimental.pallas import tpu as pltpu
from jax.experimental.pallas import tpu_sc as plsc

VOCAB = 100000
EMBED_DIM = 128
BATCH = 1024
SEQ = 200

NC = 2
NS = 16
NW = NC * NS

TOTAL = BATCH * SEQ
PER_W = TOTAL // NW
CHUNK = 128
NCH = PER_W // CHUNK
NBUF = 5
NIT = NCH // NBUF

_mesh = plsc.VectorSubcoreMesh(core_axis_name="c", subcore_axis_name="s")


@functools.partial(
    pl.kernel,
    out_type=jax.ShapeDtypeStruct((TOTAL, EMBED_DIM), jnp.float32),
    mesh=_mesh,
    scratch_types=(
        [pltpu.VMEM((NCH, CHUNK), jnp.int32)]
        + [pltpu.VMEM((CHUNK, EMBED_DIM), jnp.float32) for _ in range(NBUF)]
        + [pltpu.SemaphoreType.DMA for _ in range(2 * NBUF)]
    ),
)
def _gather_kernel(table_hbm, ids_hbm, out_hbm, idx_v, *scr):
    bufs = scr[:NBUF]
    gsem = scr[NBUF:2 * NBUF]
    ssem = scr[2 * NBUF:]

    wid = lax.axis_index("s") * NC + lax.axis_index("c")
    obase = wid * PER_W

    pltpu.sync_copy(ids_hbm.at[wid], idx_v)

    def start_gather(ch, b):
        pltpu.async_copy(table_hbm.at[idx_v.at[ch]], bufs[b], gsem[b])

    def wait_gather(b):
        pltpu.make_async_copy(table_hbm.at[pl.ds(0, CHUNK)], bufs[b],
                              gsem[b]).wait()

    def start_scatter(ch, b):
        pltpu.async_copy(bufs[b],
                         out_hbm.at[pl.ds(obase + ch * CHUNK, CHUNK)], ssem[b])

    def wait_scatter(b):
        pltpu.make_async_copy(bufs[b], out_hbm.at[pl.ds(obase, CHUNK)],
                              ssem[b]).wait()

    for b in range(NBUF):
        start_gather(b, b)

    def body(t, carry):
        base = t * NBUF
        for b in range(NBUF):
            wait_gather(b)
            start_scatter(base + b, b)
        @pl.when(t < NIT - 1)
        def _refill():
            for b in range(NBUF):
                wait_scatter(b)
                start_gather(base + NBUF + b, b)
        return carry

    lax.fori_loop(0, NIT, body, 0)
    for b in range(NBUF):
        wait_scatter(b)


def _mask_body(am_ref, out_ref):
    out_ref[...] = am_ref[...] != 1


def kernel(input_ids, attention_mask, vocab_table):
    ids = input_ids.astype(jnp.int32).reshape(NW, NCH, CHUNK)
    gathered = _gather_kernel(vocab_table, ids)
    mask = pl.pallas_call(
        _mask_body,
        out_shape=jax.ShapeDtypeStruct((BATCH, SEQ), jnp.bool_),
    )(attention_mask)
    return gathered.reshape(BATCH, SEQ, EMBED_DIM), mask
